# baseline (speedup 1.0000x reference)
import functools

import jax
import jax.numpy as jnp
from jax import lax
from jax.experimental import pallas as pl
from jax.experimental.pallas import tpu as pltpu
from jax.experimental.pallas import tpu_sc as plsc

_LANES = 16


@functools.lru_cache(maxsize=None)
def _build(batch: int, seq_len: int, hidden: int):
    info = plsc.get_sparse_core_info()
    num_workers = info.num_cores * info.num_subcores
    p = seq_len // num_workers
    assert p * num_workers == seq_len
    assert p % 8 == 0 and p <= 128
    assert hidden % _LANES == 0

    mesh = plsc.VectorSubcoreMesh(core_axis_name="c", subcore_axis_name="s")

    @functools.partial(
        pl.kernel,
        out_type=jax.ShapeDtypeStruct((batch, seq_len, hidden), jnp.float32),
        mesh=mesh,
        scratch_types=[
            pltpu.VMEM((batch, p), jnp.int32),
            pltpu.VMEM((batch, p, hidden), jnp.float32),
            pltpu.SemaphoreType.DMA,
            pltpu.SemaphoreType.DMA,
            [pltpu.SemaphoreType.DMA for _ in range(batch)],
            pltpu.SemaphoreType.DMA,
        ],
    )
    def emb_kernel(ids_hbm, table_hbm, pos_hbm, out_hbm,
                   idx_v, rows_v, sem_i, sem_p, sem_g, sem_o):
        wid = lax.axis_index("s") * info.num_cores + lax.axis_index("c")
        lo = wid * p

        idx_cps = [
            pltpu.async_copy(ids_hbm.at[bb, pl.ds(lo, p)], idx_v.at[bb], sem_i)
            for bb in range(batch)
        ]
        cp_p = pltpu.async_copy(pos_hbm.at[pl.ds(lo, p)], rows_v.at[0], sem_p)
        cp_p.wait()

        def replicate(src_b, dst_b):
            def body(r, carry):
                for c in range(hidden // _LANES):
                    sl = pl.ds(c * _LANES, _LANES)
                    rows_v[dst_b, r, sl] = rows_v[src_b, r, sl]
                return carry
            lax.fori_loop(0, p, body, 0)

        gathers = []
        for b in range(batch):
            if b + 1 < batch:
                replicate(b, b + 1)
            if b == 0:
                for cp in idx_cps:
                    cp.wait()
            gathers.append(pltpu.async_copy(
                table_hbm.at[idx_v.at[b]],
                rows_v.at[b],
                sem_g[b],
                add=True,
            ))
        for g in gathers:
            g.wait()
        pltpu.sync_copy(rows_v, out_hbm.at[:, pl.ds(lo, p)])

    return emb_kernel


def kernel(input_ids, embedding, position_embedding):
    batch, seq_len = input_ids.shape
    hidden = embedding.shape[1]
    fn = _build(batch, seq_len, hidden)
    return fn(input_ids.astype(jnp.int32), embedding, position_embedding)


# speedup vs baseline: 1.0032x; 1.0032x over previous
import functools

import jax
import jax.numpy as jnp
from jax import lax
from jax.experimental import pallas as pl
from jax.experimental.pallas import tpu as pltpu
from jax.experimental.pallas import tpu_sc as plsc

_LANES = 16


@functools.lru_cache(maxsize=None)
def _build(batch: int, seq_len: int, hidden: int):
    info = plsc.get_sparse_core_info()
    num_workers = info.num_cores * info.num_subcores
    p = seq_len // num_workers
    assert p * num_workers == seq_len
    assert p % 8 == 0 and p <= 128
    assert hidden % _LANES == 0

    mesh = plsc.VectorSubcoreMesh(core_axis_name="c", subcore_axis_name="s")

    @functools.partial(
        pl.kernel,
        out_type=jax.ShapeDtypeStruct((batch, seq_len, hidden), jnp.float32),
        mesh=mesh,
        scratch_types=[
            pltpu.VMEM((batch, p), jnp.int32),
            pltpu.VMEM((batch, p, hidden), jnp.float32),
            pltpu.SemaphoreType.DMA,
            pltpu.SemaphoreType.DMA,
            [pltpu.SemaphoreType.DMA for _ in range(batch)],
        ],
    )
    def emb_kernel(ids_hbm, table_hbm, pos_hbm, out_hbm,
                   idx_v, rows_v, sem_i, sem_p, sem_g):
        wid = lax.axis_index("s") * info.num_cores + lax.axis_index("c")
        lo = wid * p

        idx_cps = [
            pltpu.async_copy(ids_hbm.at[bb, pl.ds(lo, p)], idx_v.at[bb], sem_i)
            for bb in range(batch)
        ]
        cp_p = pltpu.async_copy(pos_hbm.at[pl.ds(lo, p)], rows_v.at[0], sem_p)
        cp_p.wait()

        def replicate(src_b, dst_b):
            def body(r, carry):
                for c in range(hidden // _LANES):
                    sl = pl.ds(c * _LANES, _LANES)
                    rows_v[dst_b, r, sl] = rows_v[src_b, r, sl]
                return carry
            lax.fori_loop(0, p, body, 0)

        gathers = []
        for b in range(batch):
            if b + 1 < batch:
                replicate(b, b + 1)
            if b == 0:
                for cp in idx_cps:
                    cp.wait()
            gathers.append(pltpu.async_copy(
                table_hbm.at[idx_v.at[b]],
                rows_v.at[b],
                sem_g[b],
                add=True,
            ))
        for g in gathers:
            g.wait()
        pltpu.sync_copy(rows_v, out_hbm.at[:, pl.ds(lo, p)])

    return emb_kernel


def kernel(input_ids, embedding, position_embedding):
    batch, seq_len = input_ids.shape
    hidden = embedding.shape[1]
    fn = _build(batch, seq_len, hidden)
    return fn(input_ids.astype(jnp.int32), embedding, position_embedding)
